# Initial kernel scaffold; baseline (speedup 1.0000x reference)
import jax
import jax.numpy as jnp
from jax.experimental import pallas as pl


def kernel(inputs, table):
    raise NotImplementedError("write your pallas kernel here")


# speedup vs baseline: 1.1088x; 1.1088x over previous
import jax
import jax.numpy as jnp
from jax import lax
from jax.experimental import pallas as pl
from jax.experimental.pallas import tpu as pltpu
from jax.experimental.pallas import tpu_sc as plsc

EMBED = 32
NC = 2
NS = 16
NW = NC * NS
IDX_MINOR = 128
G = 10
CH = G * IDX_MINOR


def _build(B, V):
    b_per_w = B // NW
    K = b_per_w // IDX_MINOR
    M = b_per_w // CH

    def body(table_hbm, idx_hbm, out_hbm, idx_v, rows_v, sem_g, sem_s0, sem_s1):
        wid = lax.axis_index("s") * NC + lax.axis_index("c")
        out0 = wid * b_per_w
        pltpu.sync_copy(idx_hbm.at[wid], idx_v)

        def chunk(m, buf, sem_s):
            @pl.when(m >= 2)
            def _():
                pltpu.make_async_copy(
                    rows_v.at[buf],
                    out_hbm.at[pl.ds(out0 + (m - 2) * CH, CH)],
                    sem_s,
                ).wait()
            handles = [
                pltpu.async_copy(
                    table_hbm.at[idx_v.at[m * G + t]],
                    rows_v.at[buf, pl.ds(t * IDX_MINOR, IDX_MINOR)],
                    sem_g,
                )
                for t in range(G)
            ]
            for h in handles:
                h.wait()
            pltpu.async_copy(
                rows_v.at[buf], out_hbm.at[pl.ds(out0 + m * CH, CH)], sem_s
            )

        def outer(mm, carry):
            chunk(mm * 2, 0, sem_s0)
            chunk(mm * 2 + 1, 1, sem_s1)
            return carry

        lax.fori_loop(0, M // 2, outer, 0)
        pltpu.make_async_copy(
            rows_v.at[0], out_hbm.at[pl.ds(out0 + (M - 2) * CH, CH)], sem_s0
        ).wait()
        pltpu.make_async_copy(
            rows_v.at[1], out_hbm.at[pl.ds(out0 + (M - 1) * CH, CH)], sem_s1
        ).wait()

    mesh = plsc.VectorSubcoreMesh(
        core_axis_name="c", subcore_axis_name="s", num_cores=NC, num_subcores=NS
    )
    return pl.kernel(
        body,
        out_type=jax.ShapeDtypeStruct((B, EMBED), jnp.float32),
        mesh=mesh,
        compiler_params=pltpu.CompilerParams(use_tc_tiling_on_sc=False),
        scratch_types=[
            pltpu.VMEM((K, IDX_MINOR), jnp.int32),
            pltpu.VMEM((2, CH, EMBED), jnp.float32),
            pltpu.SemaphoreType.DMA,
            pltpu.SemaphoreType.DMA,
            pltpu.SemaphoreType.DMA,
        ],
    )


def kernel(inputs, table):
    B = inputs.size
    idx = inputs.astype(jnp.int32).reshape(NW, B // NW // IDX_MINOR, IDX_MINOR)
    out = _build(B, table.shape[0])(table, idx)
    return out.reshape(inputs.shape + (EMBED,))


# speedup vs baseline: 1.1099x; 1.0011x over previous
import jax
import jax.numpy as jnp
from jax import lax
from jax.experimental import pallas as pl
from jax.experimental.pallas import tpu as pltpu
from jax.experimental.pallas import tpu_sc as plsc

EMBED = 32
NC = 2
NS = 16
NW = NC * NS
IDX_MINOR = 128
G = 10
CH = G * IDX_MINOR


def _build(B, V):
    b_per_w = B // NW
    K = b_per_w // IDX_MINOR
    M = b_per_w // CH

    def body(table_hbm, idx_hbm, out_hbm, idx_v, rows_v, sem_g, sem_s0, sem_s1):
        wid = lax.axis_index("s") * NC + lax.axis_index("c")
        out0 = wid * b_per_w
        pltpu.sync_copy(idx_hbm.at[wid], idx_v)

        def chunk(m, buf, sem_s):
            @pl.when(m >= 2)
            def _():
                pltpu.make_async_copy(
                    rows_v.at[buf],
                    out_hbm.at[pl.ds(out0 + (m - 2) * CH, CH)],
                    sem_s,
                ).wait()
            pltpu.async_copy(
                table_hbm.at[idx_v.at[pl.ds(m * CH, CH)]],
                rows_v.at[buf],
                sem_g,
            ).wait()
            pltpu.async_copy(
                rows_v.at[buf], out_hbm.at[pl.ds(out0 + m * CH, CH)], sem_s
            )

        def outer(mm, carry):
            chunk(mm * 2, 0, sem_s0)
            chunk(mm * 2 + 1, 1, sem_s1)
            return carry

        lax.fori_loop(0, M // 2, outer, 0)
        pltpu.make_async_copy(
            rows_v.at[0], out_hbm.at[pl.ds(out0 + (M - 2) * CH, CH)], sem_s0
        ).wait()
        pltpu.make_async_copy(
            rows_v.at[1], out_hbm.at[pl.ds(out0 + (M - 1) * CH, CH)], sem_s1
        ).wait()

    mesh = plsc.VectorSubcoreMesh(
        core_axis_name="c", subcore_axis_name="s", num_cores=NC, num_subcores=NS
    )
    return pl.kernel(
        body,
        out_type=jax.ShapeDtypeStruct((B, EMBED), jnp.float32),
        mesh=mesh,
        compiler_params=pltpu.CompilerParams(use_tc_tiling_on_sc=False),
        scratch_types=[
            pltpu.VMEM((b_per_w,), jnp.int32),
            pltpu.VMEM((2, CH, EMBED), jnp.float32),
            pltpu.SemaphoreType.DMA,
            pltpu.SemaphoreType.DMA,
            pltpu.SemaphoreType.DMA,
        ],
    )


def kernel(inputs, table):
    B = inputs.size
    idx = inputs.astype(jnp.int32).reshape(NW, B // NW)
    out = _build(B, table.shape[0])(table, idx)
    return out.reshape(inputs.shape + (EMBED,))


# speedup vs baseline: 1.5156x; 1.3655x over previous
import jax
import jax.numpy as jnp
from jax import lax
from jax.experimental import pallas as pl
from jax.experimental.pallas import tpu as pltpu
from jax.experimental.pallas import tpu_sc as plsc

EMBED = 32
NC = 2
NS = 16
NW = NC * NS
BT = 128
HIST = 50


def _build(B, V):
    n_bt = B // HIST // BT
    bt_per_w = n_bt // NW
    n_items = bt_per_w * HIST

    def body(table_hbm, idxT_hbm, x_hbm, idx_v, rows_v, tb_v,
             sg0, sg1, ss0, ss1):
        wid = lax.axis_index("s") * NC + lax.axis_index("c")
        bt0 = wid * bt_per_w

        def coords(k):
            return bt0 + k // HIST, lax.rem(k, HIST)

        def stage_in(k, buf, sem):
            bt, h = coords(k)
            pltpu.sync_copy(idxT_hbm.at[h, pl.ds(bt * BT, BT)],
                            idx_v.at[buf])
            pltpu.async_copy(table_hbm.at[idx_v.at[buf]], rows_v.at[buf],
                             sem)

        def transpose(buf):
            def e_body(e, carry):
                et = e // 8
                es = lax.rem(e, 8)
                col = jnp.zeros((16,), jnp.int32) + e
                for g in range(8):
                    rowsidx = lax.iota(jnp.int32, 16) + (16 * g)
                    v = plsc.load_gather(rows_v.at[buf], [rowsidx, col])
                    tb_v[buf, et, es, pl.ds(g * 16, 16)] = v
                return carry
            lax.fori_loop(0, EMBED, e_body, 0)

        def out_start(k, buf, sem):
            bt, h = coords(k)
            for et in range(4):
                pltpu.async_copy(tb_v.at[buf, et], x_hbm.at[h, et, bt],
                                 sem)

        def out_wait(k, buf, sem):
            bt, h = coords(k)
            for et in range(4):
                pltpu.make_async_copy(tb_v.at[buf, et],
                                      x_hbm.at[h, et, bt], sem).wait()

        def gather_wait(buf, sem):
            pltpu.make_async_copy(table_hbm.at[idx_v.at[buf]],
                                  rows_v.at[buf], sem).wait()

        stage_in(0, 0, sg0)

        def outer(kk, carry):
            k0 = 2 * kk
            stage_in(k0 + 1, 1, sg1)
            gather_wait(0, sg0)

            @pl.when(kk > 0)
            def _():
                out_wait(k0 - 2, 0, ss0)
            transpose(0)
            out_start(k0, 0, ss0)

            @pl.when(kk < n_items // 2 - 1)
            def _():
                stage_in(k0 + 2, 0, sg0)
            gather_wait(1, sg1)

            @pl.when(kk > 0)
            def _():
                out_wait(k0 - 1, 1, ss1)
            transpose(1)
            out_start(k0 + 1, 1, ss1)
            return carry

        lax.fori_loop(0, n_items // 2, outer, 0)
        out_wait(n_items - 2, 0, ss0)
        out_wait(n_items - 1, 1, ss1)

    mesh = plsc.VectorSubcoreMesh(
        core_axis_name="c", subcore_axis_name="s", num_cores=NC,
        num_subcores=NS,
    )
    return pl.kernel(
        body,
        out_type=jax.ShapeDtypeStruct(
            (HIST, EMBED // 8, n_bt, 8, BT), jnp.float32),
        mesh=mesh,
        compiler_params=pltpu.CompilerParams(
            use_tc_tiling_on_sc=False, needs_layout_passes=False
        ),
        scratch_types=[
            pltpu.VMEM((2, BT), jnp.int32),
            pltpu.VMEM((2, BT, EMBED), jnp.float32),
            pltpu.VMEM((2, EMBED // 8, 8, BT), jnp.float32),
            pltpu.SemaphoreType.DMA,
            pltpu.SemaphoreType.DMA,
            pltpu.SemaphoreType.DMA,
            pltpu.SemaphoreType.DMA,
        ],
    )


def kernel(inputs, table):
    B = inputs.size
    idxT = inputs.T.astype(jnp.int32)
    x = _build(B, table.shape[0])(table, idxT)
    out = x.transpose(2, 4, 0, 1, 3).reshape(B // HIST, HIST, EMBED)
    return out
